# baseline (speedup 1.0000x reference)
import jax
import jax.numpy as jnp
from jax.experimental import pallas as pl

NUM_EMBEDDINGS = 1024
EMBEDDING_DIM = 64
SPARSITY = 5
PATCH = 2
COMMIT = 0.25
EPS = 1e-10
ALPHA = 0.3
ATOM_DIM = EMBEDDING_DIM * PATCH * PATCH


def _gram_kernel(x_ref, dn_ref, g_ref, hbar_ref):
    X = x_ref[...]
    Dn = dn_ref[...]
    g_ref[...] = jax.lax.dot_general(
        Dn, Dn, (((0,), (0,)), ((), ())), preferred_element_type=jnp.float32)
    hbar_ref[...] = jax.lax.dot_general(
        X, Dn, (((1,), (0,)), ((), ())), preferred_element_type=jnp.float32)


def _recon_kernel(dn_ref, oh_ref, coef_ref, x_ref, recon_ref, err_ref):
    i = pl.program_id(0)
    Dn = dn_ref[...]
    X = x_ref[...]
    recon = None
    for k in range(SPARSITY):
        oh = oh_ref[k]
        d_k = jax.lax.dot_general(oh, Dn, (((1,), (1,)), ((), ())),
                                  preferred_element_type=jnp.float32)
        c_k = coef_ref[:, k:k + 1]
        term = c_k * d_k
        recon = term if recon is None else recon + term
    recon_ref[...] = recon
    diff = recon - X
    blk = jnp.sum(diff * diff, keepdims=True)

    @pl.when(i == 0)
    def _init():
        err_ref[...] = jnp.zeros_like(err_ref)

    err_ref[...] += blk


@jax.jit
def _dict_forward(z, dictionary, usage_ema):
    Bz, C, H, W = z.shape
    P = PATCH
    Hp, Wp = H // P, W // P
    patches = z.reshape(Bz, C, Hp, P, Wp, P).transpose(0, 2, 4, 1, 3, 5)
    patches = patches.reshape(Bz * Hp * Wp, C * P * P)
    X = patches
    T, M = X.shape
    N = dictionary.shape[1]

    norms = jnp.maximum(jnp.linalg.norm(dictionary, axis=0, keepdims=True), EPS)
    Dn = dictionary / norms

    G, h_bar = pl.pallas_call(
        _gram_kernel,
        out_shape=[
            jax.ShapeDtypeStruct((N, N), jnp.float32),
            jax.ShapeDtypeStruct((T, N), jnp.float32),
        ],
    )(X, Dn)

    usage = usage_ema / jnp.maximum(usage_ema.sum(), EPS)
    uniform = 1.0 / max(1.0, float(N))
    boost = jnp.minimum((uniform / jnp.maximum(usage, EPS)) ** ALPHA, 8.0)

    h = h_bar
    B = T
    L = jnp.ones((B, 1, 1), jnp.float32)
    mask = jnp.ones((B, N), dtype=bool)
    bidx = jnp.arange(B)
    I = None
    gamma_stack = None
    for k in range(1, SPARSITY + 1):
        scores = jnp.abs(h) * mask.astype(jnp.float32) * boost[None, :]
        idx = jnp.argmax(scores, axis=1)
        mask = mask.at[bidx, idx].set(False)
        if k > 1:
            G_col = G[I, idx[:, None]][..., None]
            w = jnp.linalg.solve(L, G_col)
            wT = jnp.swapaxes(w, 1, 2)
            w_corner = jnp.sqrt(jnp.maximum(
                1.0 - jnp.sum(wT ** 2, axis=2, keepdims=True), 1e-12))
            zeros = jnp.zeros((B, k - 1, 1), jnp.float32)
            L = jnp.concatenate([jnp.concatenate([L, zeros], axis=2),
                                 jnp.concatenate([wT, w_corner], axis=2)], axis=1)
            I = jnp.concatenate([I, idx[:, None]], axis=1)
        else:
            I = idx[:, None]
        h_stack = jnp.take_along_axis(h_bar, I, axis=1)[..., None]
        y = jnp.linalg.solve(L, h_stack)
        gamma_stack = jnp.linalg.solve(jnp.swapaxes(L, 1, 2), y)
        if k < SPARSITY:
            beta = jnp.einsum('bk,bkn->bn', gamma_stack[..., 0], G[I])
            h = h_bar - beta

    coeffs = gamma_stack[..., 0]
    onehots = jax.nn.one_hot(I.T, N, dtype=jnp.float32)

    block_b = 512
    recon, err = pl.pallas_call(
        _recon_kernel,
        grid=(T // block_b,),
        in_specs=[
            pl.BlockSpec((M, N), lambda i: (0, 0)),
            pl.BlockSpec((SPARSITY, block_b, N), lambda i: (0, i, 0)),
            pl.BlockSpec((block_b, SPARSITY), lambda i: (i, 0)),
            pl.BlockSpec((block_b, M), lambda i: (i, 0)),
        ],
        out_specs=[
            pl.BlockSpec((block_b, M), lambda i: (i, 0)),
            pl.BlockSpec((1, 1), lambda i: (0, 0)),
        ],
        out_shape=[
            jax.ShapeDtypeStruct((T, M), jnp.float32),
            jax.ShapeDtypeStruct((1, 1), jnp.float32),
        ],
    )(Dn, onehots, coeffs, X)

    loss = (1.0 + COMMIT) * err[0, 0] / (T * M)
    zq = recon.reshape(Bz, Hp, Wp, C, P, P).transpose(0, 3, 1, 4, 2, 5)
    zq = zq.reshape(Bz, C, H, W)
    return zq, loss


def kernel(z, dictionary, usage_ema):
    return _dict_forward(z, dictionary, usage_ema)
